# baseline (speedup 1.0000x reference)
import jax
import jax.numpy as jnp
from jax import lax
from jax.experimental import pallas as pl
from jax.experimental.pallas import tpu as pltpu
from jax.experimental.pallas import tpu_sc as plsc

N_POINTS = 524288
RES = 16
VPD = 17
N_TAB = VPD * VPD * VPD
N_TAB_PAD = 4920
F_G = 4
F_V = 8
F_OUT_G = 4
F_OUT_V = 7

NC = 2
NS = 16
NW = NC * NS
PTS_PER_W = N_POINTS // NW
CHUNK = 256
N_CHUNKS = PTS_PER_W // CHUNK
N_OUTER = N_CHUNKS // 2
GROUPS = CHUNK // 16

PIECE = 328
N_PIECES = N_TAB_PAD // PIECE
PGROUPS = 21


def _rsqrt(x):
    i = lax.bitcast_convert_type(x, jnp.int32)
    i = jnp.int32(0x5F3759DF) - lax.shift_right_logical(i, 1)
    y = lax.bitcast_convert_type(i, jnp.float32)
    for _ in range(3):
        y = y * (1.5 - 0.5 * x * y * y)
    return y


def _sc_body(in_hbm, gt_hbm, vt_hbm, go_hbm, vo_hbm, gtab_v, vtab_v,
             in_sem0, in_sem1, og_sem0, og_sem1, ov_sem0, ov_sem1):
    wid = lax.axis_index("s") * NC + lax.axis_index("c")
    lanes = lax.iota(jnp.int32, 16)
    fcols = [jnp.full((16,), f, jnp.int32) for f in range(F_V)]
    tail_mask = lanes < 8

    def _prep(tab_hbm, tab_v, f_in, f_out, tmp):
        def piece(pi, c):
            pltpu.sync_copy(tab_hbm.at[pl.ds(pi * PIECE, PIECE)], tmp)
            for k in range(PGROUPS):
                full = k < PGROUPS - 1
                m = None if full else tail_mask
                rl = jnp.minimum(k * 16 + lanes, PIECE - 1)
                rg = (pi * PIECE + rl) * f_out
                for f in range(f_out):
                    t = plsc.load_gather(tmp, [rl, fcols[f]], mask=m)
                    plsc.store_scatter(tab_v, [rg + f], t, mask=m)
            return c
        lax.fori_loop(0, N_PIECES, piece, 0)

    pl.run_scoped(
        lambda tmp: _prep(vt_hbm, vtab_v, F_V, F_OUT_V, tmp),
        pltpu.VMEM((PIECE, F_V), jnp.float32))
    pl.run_scoped(
        lambda tmp: _prep(gt_hbm, gtab_v, F_G, F_OUT_G, tmp),
        pltpu.VMEM((PIECE, F_G), jnp.float32))

    base_w = wid * PTS_PER_W

    def _main(inb0, inb1, gob0, gob1, vob0, vob1):
        inbs = (inb0, inb1)
        gobs = (gob0, gob1)
        vobs = (vob0, vob1)
        in_sems = (in_sem0, in_sem1)
        og_sems = (og_sem0, og_sem1)
        ov_sems = (ov_sem0, ov_sem1)

        for b in (0, 1):
            pltpu.async_copy(
                in_hbm.at[pl.ds(base_w + b * CHUNK, CHUNK)], inbs[b],
                in_sems[b])

        def group_body_for(inb, gob, vob):
            def group_body(gi, c2):
                s = gi * 16
                rows = s + lanes
                x = plsc.load_gather(inb, [rows, fcols[0]])
                y = plsc.load_gather(inb, [rows, fcols[1]])
                z = plsc.load_gather(inb, [rows, fcols[2]])
                px = x * jnp.float32(RES)
                py = y * jnp.float32(RES)
                pz = z * jnp.float32(RES)
                p0x = px.astype(jnp.int32)
                p0y = py.astype(jnp.int32)
                p0z = pz.astype(jnp.int32)
                fx = px - p0x.astype(jnp.float32)
                fy = py - p0y.astype(jnp.float32)
                fz = pz - p0z.astype(jnp.float32)
                zero = jnp.int32(0)
                hi = jnp.int32(RES)
                cx = (jnp.minimum(jnp.maximum(p0x, zero), hi),
                      jnp.minimum(p0x + 1, hi))
                cyo = (jnp.minimum(jnp.maximum(p0y, zero), hi) * VPD,
                       jnp.minimum(p0y + 1, hi) * VPD)
                czo = (jnp.minimum(jnp.maximum(p0z, zero), hi) * (VPD * VPD),
                       jnp.minimum(p0z + 1, hi) * (VPD * VPD))
                wx = (1.0 - fx, fx)
                wy = (1.0 - fy, fy)
                wz = (1.0 - fz, fz)

                acc = [jnp.zeros((16,), jnp.float32) for _ in range(11)]
                for dx in (0, 1):
                    for dy in (0, 1):
                        wxy = wx[dx] * wy[dy]
                        cxy = cx[dx] + cyo[dy]
                        for dz in (0, 1):
                            w = wxy * wz[dz]
                            idx = cxy + czo[dz]
                            gidx = idx * F_OUT_G
                            vidx = idx * F_OUT_V
                            for f in range(F_OUT_G):
                                t = plsc.load_gather(gtab_v, [gidx + f])
                                acc[f] = acc[f] + w * t
                            for f in range(F_OUT_V):
                                t = plsc.load_gather(vtab_v, [vidx + f])
                                acc[F_OUT_G + f] = acc[F_OUT_G + f] + w * t

                g0 = acc[0] * 50.0 + 0.5
                g1 = acc[1] * 50.0 + 0.5
                g2 = acc[2] * 50.0 + 0.5
                g3 = jnp.maximum(acc[3], 0.001)
                sharp = jnp.minimum(jnp.maximum(acc[4], 0.1), 1.0)
                a0, a1, a2 = acc[5], acc[6], acc[7]
                ss = jnp.maximum(a0 * a0 + a1 * a1 + a2 * a2, 1e-30)
                nrm = ss * _rsqrt(ss)
                den = jnp.maximum(nrm, 1e-6)
                ax0 = a0 / den
                ax1 = a1 / den
                ax2 = a2 / den
                am0 = 1.0 / (1.0 + jnp.exp(-acc[8]))
                am1 = 1.0 / (1.0 + jnp.exp(-acc[9]))
                am2 = 1.0 / (1.0 + jnp.exp(-acc[10]))

                gb = rows * F_OUT_G
                for f, val in enumerate((g0, g1, g2, g3)):
                    plsc.store_scatter(gob, [gb + f], val)
                vb = rows * F_OUT_V
                for f, val in enumerate((sharp, ax0, ax1, ax2, am0, am1, am2)):
                    plsc.store_scatter(vob, [vb + f], val)
                return c2
            return group_body

        def outer(ci2, carry):
            for b in (0, 1):
                ci = ci2 * 2 + b
                base = base_w + ci * CHUNK
                pltpu.make_async_copy(
                    in_hbm.at[pl.ds(base, CHUNK)], inbs[b], in_sems[b]).wait()

                @pl.when(ci2 > 0)
                def _wait_out():
                    pb = base - 2 * CHUNK
                    pltpu.make_async_copy(
                        gobs[b],
                        go_hbm.at[pl.ds(pb * F_OUT_G, CHUNK * F_OUT_G)],
                        og_sems[b]).wait()
                    pltpu.make_async_copy(
                        vobs[b],
                        vo_hbm.at[pl.ds(pb * F_OUT_V, CHUNK * F_OUT_V)],
                        ov_sems[b]).wait()

                lax.fori_loop(0, GROUPS,
                              group_body_for(inbs[b], gobs[b], vobs[b]), 0)

                pltpu.async_copy(
                    gobs[b], go_hbm.at[pl.ds(base * F_OUT_G, CHUNK * F_OUT_G)],
                    og_sems[b])
                pltpu.async_copy(
                    vobs[b], vo_hbm.at[pl.ds(base * F_OUT_V, CHUNK * F_OUT_V)],
                    ov_sems[b])

                @pl.when(ci2 < N_OUTER - 1)
                def _next_in():
                    pltpu.async_copy(
                        in_hbm.at[pl.ds(base + 2 * CHUNK, CHUNK)], inbs[b],
                        in_sems[b])
            return carry

        lax.fori_loop(0, N_OUTER, outer, 0)

        for b in (0, 1):
            lb = base_w + ((N_OUTER - 1) * 2 + b) * CHUNK
            pltpu.make_async_copy(
                gobs[b], go_hbm.at[pl.ds(lb * F_OUT_G, CHUNK * F_OUT_G)],
                og_sems[b]).wait()
            pltpu.make_async_copy(
                vobs[b], vo_hbm.at[pl.ds(lb * F_OUT_V, CHUNK * F_OUT_V)],
                ov_sems[b]).wait()

    pl.run_scoped(
        _main,
        pltpu.VMEM((CHUNK, 3), jnp.float32),
        pltpu.VMEM((CHUNK, 3), jnp.float32),
        pltpu.VMEM((CHUNK * F_OUT_G,), jnp.float32),
        pltpu.VMEM((CHUNK * F_OUT_G,), jnp.float32),
        pltpu.VMEM((CHUNK * F_OUT_V,), jnp.float32),
        pltpu.VMEM((CHUNK * F_OUT_V,), jnp.float32),
    )


@jax.jit
def _run(inp, gt, vt):
    mesh = plsc.VectorSubcoreMesh(core_axis_name="c", subcore_axis_name="s")
    f = pl.kernel(
        _sc_body,
        out_type=(
            jax.ShapeDtypeStruct((N_POINTS * F_OUT_G,), jnp.float32),
            jax.ShapeDtypeStruct((N_POINTS * F_OUT_V,), jnp.float32),
        ),
        mesh=mesh,
        compiler_params=pltpu.CompilerParams(needs_layout_passes=False),
        scratch_types=[
            pltpu.VMEM((N_TAB_PAD * F_OUT_G,), jnp.float32),
            pltpu.VMEM((N_TAB_PAD * F_OUT_V,), jnp.float32),
            pltpu.SemaphoreType.DMA,
            pltpu.SemaphoreType.DMA,
            pltpu.SemaphoreType.DMA,
            pltpu.SemaphoreType.DMA,
            pltpu.SemaphoreType.DMA,
            pltpu.SemaphoreType.DMA,
        ],
    )
    return f(inp, gt, vt)


def kernel(input, gaussian_table, vmf_table):
    go, vo = _run(input, gaussian_table, vmf_table)
    return (go.reshape(N_POINTS, F_OUT_G), vo.reshape(N_POINTS, F_OUT_V))
